# baseline (speedup 1.0000x reference)
import functools

import jax
import jax.numpy as jnp
from jax import lax
from jax.experimental import pallas as pl
from jax.experimental.pallas import tpu as pltpu
from jax.experimental.pallas import tpu_sc as plsc

N_POINTS = 16384
N_COMP = 8192
L = 16
IW = 128


def _rsqrt(v):
    yi = jnp.int32(0x5F3759DF) - (plsc.bitcast(v, jnp.int32) >> 1)
    y = plsc.bitcast(yi, jnp.float32)
    for _ in range(3):
        y = y * (1.5 - 0.5 * v * y * y)
    return y


def _erfc(x):
    t = 1.0 / (1.0 + 0.5 * x)
    p = jnp.float32(0.17087277)
    for c in (-0.82215223, 1.48851587, -1.13520398, 0.27886807,
              -0.18628806, 0.09678418, 0.37409196, 1.00002368):
        p = jnp.float32(c) + t * p
    return t * jnp.exp(-x * x - 1.26551223 + t * p)


def _log(x):
    xi = plsc.bitcast(x, jnp.int32)
    ex = (xi >> 23) - 127
    m = plsc.bitcast((xi & jnp.int32(0x007FFFFF)) | jnp.int32(0x3F800000),
                     jnp.float32)
    adj = m > 1.4142135
    m = jnp.where(adj, 0.5 * m, m)
    ef = ex.astype(jnp.float32) + jnp.where(adj, 1.0, 0.0)
    s = (m - 1.0) / (m + 1.0)
    s2 = s * s
    lnm = 2.0 * s * (1.0 + s2 * (1.0 / 3.0 + s2 * (0.2 + s2 * (1.0 / 7.0))))
    return ef * 0.6931471805599453 + lnm


def _make_sc_call():
    info = plsc.get_sparse_core_info()
    nc, ns = info.num_cores, info.num_subcores
    nw = nc * ns
    assert N_COMP % (nw * L) == 0
    chunk = N_COMP // nw
    groups = chunk // L
    kk = chunk // IW
    per_row = IW // L
    mesh = plsc.VectorSubcoreMesh(core_axis_name="c", subcore_axis_name="s")

    @functools.partial(
        pl.kernel,
        out_type=jax.ShapeDtypeStruct((nc, IW), jnp.float32),
        mesh=mesh,
        compiler_params=pltpu.CompilerParams(needs_layout_passes=False,
                                             skip_device_barrier=True),
        scratch_types=[
            pltpu.VMEM((chunk,), jnp.int32),
            pltpu.VMEM((chunk,), jnp.int32),
            pltpu.VMEM((chunk,), jnp.float32),
            pltpu.VMEM((chunk,), jnp.float32),
            pltpu.VMEM((chunk,), jnp.float32),
            pltpu.VMEM((chunk,), jnp.float32),
            pltpu.VMEM((chunk,), jnp.float32),
            pltpu.VMEM((IW,), jnp.float32),
            pltpu.VMEM_SHARED((ns, IW), jnp.float32),
            pltpu.VMEM((ns, IW), jnp.float32),
            pltpu.SemaphoreType.DMA,
            pltpu.SemaphoreType.DMA,
        ],
    )
    def call(wl_hbm, mean_hbm, var_hbm, cw_hbm, out,
             wi_v, li_v, mw_v, ml_v, vw_v, vl_v, cw_v,
             st_v, shared_v, red_v, sem_a, sem_b):
        cid = lax.axis_index("c")
        sid = lax.axis_index("s")
        wid = sid * nc + cid
        base = wid * chunk

        stage = [
            pltpu.async_copy(wl_hbm.at[pl.ds(base, chunk)], wi_v, sem_a),
            pltpu.async_copy(wl_hbm.at[pl.ds(N_COMP + base, chunk)], li_v, sem_a),
            pltpu.async_copy(cw_hbm.at[pl.ds(base, chunk)], cw_v, sem_a),
        ]
        for c in stage:
            c.wait()

        sems = [sem_a, sem_b]
        gathers = [[] for _ in range(kk)]
        for k in range(kk):
            s = pl.ds(IW * k, IW)
            gathers[k].append(pltpu.async_copy(
                mean_hbm.at[wi_v.at[s]], mw_v.at[s], sems[k % 2]))
            gathers[k].append(pltpu.async_copy(
                mean_hbm.at[li_v.at[s]], ml_v.at[s], sems[k % 2]))
            gathers[k].append(pltpu.async_copy(
                var_hbm.at[wi_v.at[s]], vw_v.at[s], sems[k % 2]))
            gathers[k].append(pltpu.async_copy(
                var_hbm.at[li_v.at[s]], vl_v.at[s], sems[k % 2]))

        def group_body(i, carry):
            acc, wacc = carry
            g = pl.ds(L * i, L)
            mw = mw_v[g]
            ml = ml_v[g]
            vw = vw_v[g]
            vl = vl_v[g]
            cw = cw_v[g]

            md = mw - ml
            vd = vw + vl + 1e-6
            w = md * _rsqrt(vd) * 0.7071067811865476
            erf_abs = 1.0 - _erfc(jnp.abs(w))
            erf_s = jnp.where(w >= 0, erf_abs, -erf_abs)
            cdf = 0.5 * (1.0 + erf_s)
            lp = _log(cdf + 1e-8)
            return acc + cw * lp, wacc + cw

        for g in gathers:
            for c in g:
                c.wait()
        acc, wacc = lax.fori_loop(
            0, groups, group_body,
            (jnp.zeros((L,), jnp.float32), jnp.zeros((L,), jnp.float32)))

        st_v[pl.ds(0, L)] = acc
        st_v[pl.ds(L, L)] = wacc
        pltpu.sync_copy(st_v, shared_v.at[sid])
        plsc.subcore_barrier()

        @pl.when(sid == 0)
        def _():
            pltpu.sync_copy(shared_v, red_v)

            def red_body(r, carry):
                tot, wtot = carry
                return tot + red_v[r, pl.ds(0, L)], wtot + red_v[r, pl.ds(L, L)]

            tot, wtot = lax.fori_loop(
                0, ns, red_body,
                (jnp.zeros((L,), jnp.float32), jnp.zeros((L,), jnp.float32)))
            st_v[pl.ds(0, L)] = tot
            st_v[pl.ds(L, L)] = wtot
            pltpu.sync_copy(st_v, out.at[cid])

    return call


_sc_call = None


def kernel(mean, variance, target, confidence_weights):
    global _sc_call
    if _sc_call is None:
        _sc_call = _make_sc_call()
    tgt = target.astype(jnp.int32)
    wl = jnp.concatenate([tgt[:, 0], tgt[:, 1]])
    out = _sc_call(wl, mean, variance, confidence_weights)
    s1 = jnp.sum(out[:, :L])
    ws = jnp.sum(out[:, L:2 * L])
    n = jnp.float32(confidence_weights.shape[0])
    return jnp.where(ws > 0, s1 * (n / ws), s1)


# speedup vs baseline: 1.0056x; 1.0056x over previous
import functools

import jax
import jax.numpy as jnp
from jax import lax
from jax.experimental import pallas as pl
from jax.experimental.pallas import tpu as pltpu
from jax.experimental.pallas import tpu_sc as plsc

N_POINTS = 16384
N_COMP = 8192
L = 16
IW = 128


def _rsqrt(v):
    yi = jnp.int32(0x5F3759DF) - (plsc.bitcast(v, jnp.int32) >> 1)
    y = plsc.bitcast(yi, jnp.float32)
    for _ in range(3):
        y = y * (1.5 - 0.5 * v * y * y)
    return y


def _erfc(x):
    t = 1.0 / (1.0 + 0.5 * x)
    p = jnp.float32(0.17087277)
    for c in (-0.82215223, 1.48851587, -1.13520398, 0.27886807,
              -0.18628806, 0.09678418, 0.37409196, 1.00002368):
        p = jnp.float32(c) + t * p
    return t * jnp.exp(-x * x - 1.26551223 + t * p)


def _log(x):
    xi = plsc.bitcast(x, jnp.int32)
    ex = (xi >> 23) - 127
    m = plsc.bitcast((xi & jnp.int32(0x007FFFFF)) | jnp.int32(0x3F800000),
                     jnp.float32)
    adj = m > 1.4142135
    m = jnp.where(adj, 0.5 * m, m)
    ef = ex.astype(jnp.float32) + jnp.where(adj, 1.0, 0.0)
    s = (m - 1.0) / (m + 1.0)
    s2 = s * s
    lnm = 2.0 * s * (1.0 + s2 * (1.0 / 3.0 + s2 * (0.2 + s2 * (1.0 / 7.0))))
    return ef * 0.6931471805599453 + lnm


def _make_sc_call():
    info = plsc.get_sparse_core_info()
    nc, ns = info.num_cores, info.num_subcores
    nw = nc * ns
    assert N_COMP % (nw * L) == 0
    chunk = N_COMP // nw
    groups = chunk // L
    kk = chunk // IW
    per_row = IW // L
    mesh = plsc.VectorSubcoreMesh(core_axis_name="c", subcore_axis_name="s")

    @functools.partial(
        pl.kernel,
        out_type=jax.ShapeDtypeStruct((nc, IW), jnp.float32),
        mesh=mesh,
        compiler_params=pltpu.CompilerParams(needs_layout_passes=False),
        scratch_types=[
            pltpu.VMEM((chunk,), jnp.int32),
            pltpu.VMEM((chunk,), jnp.int32),
            pltpu.VMEM((chunk,), jnp.float32),
            pltpu.VMEM((chunk,), jnp.float32),
            pltpu.VMEM((chunk,), jnp.float32),
            pltpu.VMEM((chunk,), jnp.float32),
            pltpu.VMEM((chunk,), jnp.float32),
            pltpu.VMEM((IW,), jnp.float32),
            pltpu.VMEM_SHARED((ns, IW), jnp.float32),
            pltpu.VMEM((ns, IW), jnp.float32),
            pltpu.SemaphoreType.DMA,
            pltpu.SemaphoreType.DMA,
        ],
    )
    def call(wl_hbm, mean_hbm, var_hbm, cw_hbm, out,
             wi_v, li_v, mw_v, ml_v, vw_v, vl_v, cw_v,
             st_v, shared_v, red_v, sem_a, sem_b):
        cid = lax.axis_index("c")
        sid = lax.axis_index("s")
        wid = sid * nc + cid
        base = wid * chunk

        stage = [
            pltpu.async_copy(wl_hbm.at[pl.ds(base, chunk)], wi_v, sem_a),
            pltpu.async_copy(wl_hbm.at[pl.ds(N_COMP + base, chunk)], li_v, sem_a),
            pltpu.async_copy(cw_hbm.at[pl.ds(base, chunk)], cw_v, sem_a),
        ]
        for c in stage:
            c.wait()

        sems = [sem_a, sem_b]
        gathers = [[] for _ in range(kk)]
        for k in range(kk):
            s = pl.ds(IW * k, IW)
            gathers[k].append(pltpu.async_copy(
                mean_hbm.at[wi_v.at[s]], mw_v.at[s], sems[k % 2]))
            gathers[k].append(pltpu.async_copy(
                mean_hbm.at[li_v.at[s]], ml_v.at[s], sems[k % 2]))
            gathers[k].append(pltpu.async_copy(
                var_hbm.at[wi_v.at[s]], vw_v.at[s], sems[k % 2]))
            gathers[k].append(pltpu.async_copy(
                var_hbm.at[li_v.at[s]], vl_v.at[s], sems[k % 2]))

        def group_body(i, carry):
            acc, wacc = carry
            g = pl.ds(L * i, L)
            mw = mw_v[g]
            ml = ml_v[g]
            vw = vw_v[g]
            vl = vl_v[g]
            cw = cw_v[g]

            md = mw - ml
            vd = vw + vl + 1e-6
            w = md * _rsqrt(vd) * 0.7071067811865476
            erf_abs = 1.0 - _erfc(jnp.abs(w))
            erf_s = jnp.where(w >= 0, erf_abs, -erf_abs)
            cdf = 0.5 * (1.0 + erf_s)
            lp = _log(cdf + 1e-8)
            return acc + cw * lp, wacc + cw

        for g in gathers:
            for c in g:
                c.wait()
        acc, wacc = lax.fori_loop(
            0, groups, group_body,
            (jnp.zeros((L,), jnp.float32), jnp.zeros((L,), jnp.float32)))

        st_v[pl.ds(0, L)] = acc
        st_v[pl.ds(L, L)] = wacc
        pltpu.sync_copy(st_v, shared_v.at[sid])
        plsc.subcore_barrier()

        @pl.when(sid == 0)
        def _():
            pltpu.sync_copy(shared_v, red_v)

            def red_body(r, carry):
                tot, wtot = carry
                return tot + red_v[r, pl.ds(0, L)], wtot + red_v[r, pl.ds(L, L)]

            tot, wtot = lax.fori_loop(
                0, ns, red_body,
                (jnp.zeros((L,), jnp.float32), jnp.zeros((L,), jnp.float32)))
            st_v[pl.ds(0, L)] = tot
            st_v[pl.ds(L, L)] = wtot
            pltpu.sync_copy(st_v, out.at[cid])

    return call


_sc_call = None


def kernel(mean, variance, target, confidence_weights):
    global _sc_call
    if _sc_call is None:
        _sc_call = _make_sc_call()
    tgt = target.astype(jnp.int32)
    wl = tgt.T.reshape(-1)
    out = _sc_call(wl, mean, variance, confidence_weights)
    s1 = jnp.sum(out[:, :L])
    ws = jnp.sum(out[:, L:2 * L])
    n = jnp.float32(confidence_weights.shape[0])
    return jnp.where(ws > 0, s1 * (n / ws), s1)
